# Initial kernel scaffold; baseline (speedup 1.0000x reference)
import functools

import jax
import jax.numpy as jnp
from jax import lax
from jax.experimental import pallas as pl
from jax.experimental.pallas import tpu as pltpu
from jax.experimental.pallas import tpu_sc as plsc

N = 10000
N_PAD = 10240
IN_DIM = 128
HID = 32
OUT_DIM = 2
NC = 2
NS = 16
CHUNK = 128
NCHUNK = 80
E_PAD = NC * NS * NCHUNK * CHUNK
ROWS_PER_TILE = N_PAD // NS

_F32 = jnp.float32


def _spmm_body(h_hbm, src_hbm, dst_hbm, out_hbm, src_v, dst_v, rows_v, zrow_v,
               acc, sem):
    c = lax.axis_index("c")
    s = lax.axis_index("s")
    pltpu.sync_copy(src_hbm.at[c, s], src_v)
    pltpu.sync_copy(dst_hbm.at[c, s], dst_v)
    zero16 = jnp.zeros((16,), _F32)

    def _z(i, carry):
        zrow_v[i, pl.ds(0, 16)] = zero16
        zrow_v[i, pl.ds(16, 16)] = zero16
        return carry

    lax.fori_loop(0, CHUNK, _z, 0)
    for k in range(ROWS_PER_TILE // CHUNK):
        pltpu.sync_copy(zrow_v, acc.at[pl.ds(s * ROWS_PER_TILE + k * CHUNK, CHUNK)])
    plsc.subcore_barrier()

    def _edge_chunk(j, carry):
        pltpu.async_copy(h_hbm.at[src_v.at[j]], rows_v, sem).wait()
        pltpu.sync_copy(rows_v, acc.at[dst_v.at[j]], add=True)
        return carry

    lax.fori_loop(0, NCHUNK, _edge_chunk, 0)
    plsc.subcore_barrier()
    pltpu.sync_copy(acc.at[pl.ds(s * ROWS_PER_TILE, ROWS_PER_TILE)],
                    out_hbm.at[c, pl.ds(s * ROWS_PER_TILE, ROWS_PER_TILE)])


def _deg_body(dst_hbm, out_hbm, dst_v, ones_v, zbuf_v, acc, sem):
    c = lax.axis_index("c")
    s = lax.axis_index("s")
    pltpu.sync_copy(dst_hbm.at[c, s], dst_v)
    one16 = jnp.full((16,), 1.0, _F32)
    for i in range(CHUNK // 16):
        ones_v[pl.ds(i * 16, 16)] = one16
    zero16 = jnp.zeros((16,), _F32)

    def _z(i, carry):
        zbuf_v[pl.ds(i * 16, 16)] = zero16
        return carry

    lax.fori_loop(0, ROWS_PER_TILE // 16, _z, 0)
    pltpu.sync_copy(zbuf_v, acc.at[pl.ds(s * ROWS_PER_TILE, ROWS_PER_TILE)])
    plsc.subcore_barrier()

    def _edge_chunk(j, carry):
        pltpu.sync_copy(ones_v, acc.at[dst_v.at[j]], add=True)
        return carry

    lax.fori_loop(0, NCHUNK, _edge_chunk, 0)
    plsc.subcore_barrier()
    pltpu.sync_copy(acc.at[pl.ds(s * ROWS_PER_TILE, ROWS_PER_TILE)],
                    out_hbm.at[c, pl.ds(s * ROWS_PER_TILE, ROWS_PER_TILE)])


def _make_spmm():
    mesh = plsc.VectorSubcoreMesh(core_axis_name="c", subcore_axis_name="s")
    return pl.kernel(
        _spmm_body,
        out_type=jax.ShapeDtypeStruct((NC, N_PAD, HID), _F32),
        mesh=mesh,
        scratch_types=[
            pltpu.VMEM((NCHUNK, CHUNK), jnp.int32),
            pltpu.VMEM((NCHUNK, CHUNK), jnp.int32),
            pltpu.VMEM((CHUNK, HID), _F32),
            pltpu.VMEM((CHUNK, HID), _F32),
            pltpu.VMEM_SHARED((N_PAD, HID), _F32),
            pltpu.SemaphoreType.DMA,
        ],
        name="gcn_spmm_sc",
    )


def _make_deg():
    mesh = plsc.VectorSubcoreMesh(core_axis_name="c", subcore_axis_name="s")
    return pl.kernel(
        _deg_body,
        out_type=jax.ShapeDtypeStruct((NC, N_PAD), _F32),
        mesh=mesh,
        scratch_types=[
            pltpu.VMEM((NCHUNK, CHUNK), jnp.int32),
            pltpu.VMEM((CHUNK,), _F32),
            pltpu.VMEM((ROWS_PER_TILE,), _F32),
            pltpu.VMEM_SHARED((N_PAD,), _F32),
            pltpu.SemaphoreType.DMA,
        ],
        name="gcn_deg_sc",
    )


_BLK = 1280


def _tc0_body(deg_ref, x_ref, w_ref, dinv_ref, h_ref):
    deg = deg_ref[:, 0:1] + deg_ref[:, 1:2] + 1.0
    dinv = lax.rsqrt(deg)
    dinv32 = jnp.broadcast_to(dinv, (dinv.shape[0], HID))
    dinv_ref[...] = dinv32
    h_ref[...] = dinv32 * jnp.dot(x_ref[...], w_ref[...],
                                  preferred_element_type=_F32)


def _tc_mid_body(agg_ref, h_ref, dinv_ref, w_ref, b_ref, out_ref):
    a = agg_ref[0] + agg_ref[1] + h_ref[...]
    xn = jnp.maximum(dinv_ref[...] * a + b_ref[...], 0.0)
    out_ref[...] = dinv_ref[...] * jnp.dot(xn, w_ref[...],
                                           preferred_element_type=_F32)


def _tc_last_body(agg_ref, h_ref, dinv_ref, w_ref, b_ref, out_ref):
    a = dinv_ref[...] * (agg_ref[0] + agg_ref[1] + h_ref[...])
    out_ref[...] = jnp.dot(a, w_ref[...], preferred_element_type=_F32) + b_ref[...]


def _tc0(degT, x_pad, W1):
    grid = (N_PAD // _BLK,)
    return pl.pallas_call(
        _tc0_body,
        grid=grid,
        in_specs=[
            pl.BlockSpec((_BLK, NC), lambda i: (i, 0)),
            pl.BlockSpec((_BLK, IN_DIM), lambda i: (i, 0)),
            pl.BlockSpec((IN_DIM, HID), lambda i: (0, 0)),
        ],
        out_specs=[
            pl.BlockSpec((_BLK, HID), lambda i: (i, 0)),
            pl.BlockSpec((_BLK, HID), lambda i: (i, 0)),
        ],
        out_shape=[
            jax.ShapeDtypeStruct((N_PAD, HID), _F32),
            jax.ShapeDtypeStruct((N_PAD, HID), _F32),
        ],
        name="gcn_tc0",
    )(degT, x_pad, W1)


def _tc_mid(agg, h, dinv32, W, b):
    grid = (N_PAD // _BLK,)
    return pl.pallas_call(
        _tc_mid_body,
        grid=grid,
        in_specs=[
            pl.BlockSpec((NC, _BLK, HID), lambda i: (0, i, 0)),
            pl.BlockSpec((_BLK, HID), lambda i: (i, 0)),
            pl.BlockSpec((_BLK, HID), lambda i: (i, 0)),
            pl.BlockSpec((HID, HID), lambda i: (0, 0)),
            pl.BlockSpec((1, HID), lambda i: (0, 0)),
        ],
        out_specs=pl.BlockSpec((_BLK, HID), lambda i: (i, 0)),
        out_shape=jax.ShapeDtypeStruct((N_PAD, HID), _F32),
        name="gcn_tc_mid",
    )(agg, h, dinv32, W, b)


def _tc_last(agg, h, dinv32, W4, b4):
    grid = (N_PAD // _BLK,)
    return pl.pallas_call(
        _tc_last_body,
        grid=grid,
        in_specs=[
            pl.BlockSpec((NC, _BLK, HID), lambda i: (0, i, 0)),
            pl.BlockSpec((_BLK, HID), lambda i: (i, 0)),
            pl.BlockSpec((_BLK, HID), lambda i: (i, 0)),
            pl.BlockSpec((HID, OUT_DIM), lambda i: (0, 0)),
            pl.BlockSpec((1, OUT_DIM), lambda i: (0, 0)),
        ],
        out_specs=pl.BlockSpec((_BLK, OUT_DIM), lambda i: (i, 0)),
        out_shape=jax.ShapeDtypeStruct((N_PAD, OUT_DIM), _F32),
        name="gcn_tc_last",
    )(agg, h, dinv32, W4, b4)


def kernel(x, edge_index, W1, b1, W2, b2, W3, b3, W4, b4):
    src = edge_index[0].astype(jnp.int32)
    dst = edge_index[1].astype(jnp.int32)
    pad = E_PAD - src.shape[0]
    src_p = jnp.concatenate([src, jnp.zeros((pad,), jnp.int32)])
    dst_p = jnp.concatenate([dst, jnp.full((pad,), N, jnp.int32)])
    src_p = src_p.reshape(NC, NS, NCHUNK, CHUNK)
    dst_p = dst_p.reshape(NC, NS, NCHUNK, CHUNK)
    x_pad = jnp.pad(x, ((0, N_PAD - N), (0, 0)))

    spmm = _make_spmm()
    degp = _make_deg()(dst_p)
    degT = degp.T

    dinv32, h1 = _tc0(degT, x_pad, W1)
    agg1 = spmm(h1, src_p, dst_p)
    h2 = _tc_mid(agg1, h1, dinv32, W2, b1.reshape(1, HID))
    agg2 = spmm(h2, src_p, dst_p)
    h3 = _tc_mid(agg2, h2, dinv32, W3, b2.reshape(1, HID))
    agg3 = spmm(h3, src_p, dst_p)
    x4 = _tc_mid(agg3, h3, dinv32, jnp.eye(HID, dtype=_F32),
                 b3.reshape(1, HID))
    agg4 = spmm(x4, src_p, dst_p)
    out = _tc_last(agg4, x4, dinv32, W4, b4.reshape(1, OUT_DIM))
    return out[:N]


# speedup vs baseline: 18.0976x; 18.0976x over previous
import functools

import jax
import jax.numpy as jnp
from jax import lax
from jax.experimental import pallas as pl
from jax.experimental.pallas import tpu as pltpu
from jax.experimental.pallas import tpu_sc as plsc

N = 10000
N_PAD = 10240
IN_DIM = 128
HID = 32
OUT_DIM = 2
NC = 2
NS = 16
CHUNK = 128
NCHUNK = 80
E_PAD = NC * NS * NCHUNK * CHUNK
ROWS_PER_TILE = N_PAD // NS

_F32 = jnp.float32


def _spmm_body(h_hbm, src_hbm, dst_hbm, out_hbm, src_v, dst_v, rows_v, zrow_v,
               acc, sem):
    c = lax.axis_index("c")
    s = lax.axis_index("s")
    pltpu.sync_copy(src_hbm.at[c, s], src_v)
    pltpu.sync_copy(dst_hbm.at[c, s], dst_v)
    zero16 = jnp.zeros((16,), _F32)

    def _z(i, carry):
        zrow_v[i, pl.ds(0, 16)] = zero16
        zrow_v[i, pl.ds(16, 16)] = zero16
        return carry

    lax.fori_loop(0, CHUNK, _z, 0)
    for k in range(ROWS_PER_TILE // CHUNK):
        pltpu.sync_copy(zrow_v, acc.at[pl.ds(s * ROWS_PER_TILE + k * CHUNK, CHUNK)])
    plsc.subcore_barrier()

    def _edge_chunk(j, carry):
        pltpu.async_copy(h_hbm.at[src_v.at[j]], rows_v, sem).wait()
        pltpu.sync_copy(rows_v, acc.at[dst_v.at[j]], add=True)
        return carry

    lax.fori_loop(0, NCHUNK, _edge_chunk, 0)
    plsc.subcore_barrier()
    pltpu.sync_copy(acc.at[pl.ds(s * ROWS_PER_TILE, ROWS_PER_TILE)],
                    out_hbm.at[c, pl.ds(s * ROWS_PER_TILE, ROWS_PER_TILE)])


def _deg_body(dst_hbm, out_hbm, dst_v, ones_v, zbuf_v, acc, sem):
    c = lax.axis_index("c")
    s = lax.axis_index("s")
    pltpu.sync_copy(dst_hbm.at[c, s], dst_v)
    one16 = jnp.full((16,), 1.0, _F32)
    for i in range(CHUNK // 16):
        ones_v[pl.ds(i * 16, 16)] = one16
    zero16 = jnp.zeros((16,), _F32)

    def _z(i, carry):
        zbuf_v[pl.ds(i * 16, 16)] = zero16
        return carry

    lax.fori_loop(0, ROWS_PER_TILE // 16, _z, 0)
    pltpu.sync_copy(zbuf_v, acc.at[pl.ds(s * ROWS_PER_TILE, ROWS_PER_TILE)])
    plsc.subcore_barrier()

    def _edge_chunk(j, carry):
        pltpu.sync_copy(ones_v, acc.at[dst_v.at[j]], add=True)
        return carry

    lax.fori_loop(0, NCHUNK, _edge_chunk, 0)
    plsc.subcore_barrier()
    pltpu.sync_copy(acc.at[pl.ds(s * ROWS_PER_TILE, ROWS_PER_TILE)],
                    out_hbm.at[c, pl.ds(s * ROWS_PER_TILE, ROWS_PER_TILE)])


def _make_spmm():
    mesh = plsc.VectorSubcoreMesh(core_axis_name="c", subcore_axis_name="s")
    return pl.kernel(
        _spmm_body,
        out_type=jax.ShapeDtypeStruct((NC, N_PAD, HID), _F32),
        mesh=mesh,
        scratch_types=[
            pltpu.VMEM((NCHUNK, CHUNK), jnp.int32),
            pltpu.VMEM((NCHUNK, CHUNK), jnp.int32),
            pltpu.VMEM((CHUNK, HID), _F32),
            pltpu.VMEM((CHUNK, HID), _F32),
            pltpu.VMEM_SHARED((N_PAD, HID), _F32),
            pltpu.SemaphoreType.DMA,
        ],
        compiler_params=pltpu.CompilerParams(use_tc_tiling_on_sc=False),
        name="gcn_spmm_sc",
    )


def _make_deg():
    mesh = plsc.VectorSubcoreMesh(core_axis_name="c", subcore_axis_name="s")
    return pl.kernel(
        _deg_body,
        out_type=jax.ShapeDtypeStruct((NC, N_PAD), _F32),
        mesh=mesh,
        scratch_types=[
            pltpu.VMEM((NCHUNK, CHUNK), jnp.int32),
            pltpu.VMEM((CHUNK,), _F32),
            pltpu.VMEM((ROWS_PER_TILE,), _F32),
            pltpu.VMEM_SHARED((N_PAD,), _F32),
            pltpu.SemaphoreType.DMA,
        ],
        compiler_params=pltpu.CompilerParams(use_tc_tiling_on_sc=False),
        name="gcn_deg_sc",
    )


_BLK = 1280


def _tc0_body(deg_ref, x_ref, w_ref, dinv_ref, h_ref):
    deg = deg_ref[:, 0:1] + deg_ref[:, 1:2] + 1.0
    dinv = lax.rsqrt(deg)
    dinv32 = jnp.broadcast_to(dinv, (dinv.shape[0], HID))
    dinv_ref[...] = dinv32
    h_ref[...] = dinv32 * jnp.dot(x_ref[...], w_ref[...],
                                  preferred_element_type=_F32)


def _tc_mid_body(agg_ref, h_ref, dinv_ref, w_ref, b_ref, out_ref):
    a = agg_ref[0] + agg_ref[1] + h_ref[...]
    xn = jnp.maximum(dinv_ref[...] * a + b_ref[...], 0.0)
    out_ref[...] = dinv_ref[...] * jnp.dot(xn, w_ref[...],
                                           preferred_element_type=_F32)


def _tc_last_body(agg_ref, h_ref, dinv_ref, w_ref, b_ref, out_ref):
    a = dinv_ref[...] * (agg_ref[0] + agg_ref[1] + h_ref[...])
    out_ref[...] = jnp.dot(a, w_ref[...], preferred_element_type=_F32) + b_ref[...]


def _tc0(degT, x_pad, W1):
    grid = (N_PAD // _BLK,)
    return pl.pallas_call(
        _tc0_body,
        grid=grid,
        in_specs=[
            pl.BlockSpec((_BLK, NC), lambda i: (i, 0)),
            pl.BlockSpec((_BLK, IN_DIM), lambda i: (i, 0)),
            pl.BlockSpec((IN_DIM, HID), lambda i: (0, 0)),
        ],
        out_specs=[
            pl.BlockSpec((_BLK, HID), lambda i: (i, 0)),
            pl.BlockSpec((_BLK, HID), lambda i: (i, 0)),
        ],
        out_shape=[
            jax.ShapeDtypeStruct((N_PAD, HID), _F32),
            jax.ShapeDtypeStruct((N_PAD, HID), _F32),
        ],
        name="gcn_tc0",
    )(degT, x_pad, W1)


def _tc_mid(agg, h, dinv32, W, b):
    grid = (N_PAD // _BLK,)
    return pl.pallas_call(
        _tc_mid_body,
        grid=grid,
        in_specs=[
            pl.BlockSpec((NC, _BLK, HID), lambda i: (0, i, 0)),
            pl.BlockSpec((_BLK, HID), lambda i: (i, 0)),
            pl.BlockSpec((_BLK, HID), lambda i: (i, 0)),
            pl.BlockSpec((HID, HID), lambda i: (0, 0)),
            pl.BlockSpec((1, HID), lambda i: (0, 0)),
        ],
        out_specs=pl.BlockSpec((_BLK, HID), lambda i: (i, 0)),
        out_shape=jax.ShapeDtypeStruct((N_PAD, HID), _F32),
        name="gcn_tc_mid",
    )(agg, h, dinv32, W, b)


def _tc_last(agg, h, dinv32, W4, b4):
    grid = (N_PAD // _BLK,)
    return pl.pallas_call(
        _tc_last_body,
        grid=grid,
        in_specs=[
            pl.BlockSpec((NC, _BLK, HID), lambda i: (0, i, 0)),
            pl.BlockSpec((_BLK, HID), lambda i: (i, 0)),
            pl.BlockSpec((_BLK, HID), lambda i: (i, 0)),
            pl.BlockSpec((HID, OUT_DIM), lambda i: (0, 0)),
            pl.BlockSpec((1, OUT_DIM), lambda i: (0, 0)),
        ],
        out_specs=pl.BlockSpec((_BLK, OUT_DIM), lambda i: (i, 0)),
        out_shape=jax.ShapeDtypeStruct((N_PAD, OUT_DIM), _F32),
        name="gcn_tc_last",
    )(agg, h, dinv32, W4, b4)


def kernel(x, edge_index, W1, b1, W2, b2, W3, b3, W4, b4):
    src = edge_index[0].astype(jnp.int32)
    dst = edge_index[1].astype(jnp.int32)
    pad = E_PAD - src.shape[0]
    src_p = jnp.concatenate([src, jnp.zeros((pad,), jnp.int32)])
    dst_p = jnp.concatenate([dst, jnp.full((pad,), N, jnp.int32)])
    src_p = src_p.reshape(NC, NS, NCHUNK, CHUNK)
    dst_p = dst_p.reshape(NC, NS, NCHUNK, CHUNK)
    x_pad = jnp.pad(x, ((0, N_PAD - N), (0, 0)))

    spmm = _make_spmm()
    degp = _make_deg()(dst_p)
    degT = degp.T

    dinv32, h1 = _tc0(degT, x_pad, W1)
    agg1 = spmm(h1, src_p, dst_p)
    h2 = _tc_mid(agg1, h1, dinv32, W2, b1.reshape(1, HID))
    agg2 = spmm(h2, src_p, dst_p)
    h3 = _tc_mid(agg2, h2, dinv32, W3, b2.reshape(1, HID))
    agg3 = spmm(h3, src_p, dst_p)
    x4 = _tc_mid(agg3, h3, dinv32, jnp.eye(HID, dtype=_F32),
                 b3.reshape(1, HID))
    agg4 = spmm(x4, src_p, dst_p)
    out = _tc_last(agg4, x4, dinv32, W4, b4.reshape(1, OUT_DIM))
    return out[:N]


# speedup vs baseline: 22.9964x; 1.2707x over previous
import functools

import jax
import jax.numpy as jnp
from jax import lax
from jax.experimental import pallas as pl
from jax.experimental.pallas import tpu as pltpu
from jax.experimental.pallas import tpu_sc as plsc

N = 10000
N_PAD = 10240
IN_DIM = 128
HID = 32
OUT_DIM = 2
NC = 2
NS = 16
CHUNK = 128
NCHUNK = 80
NBUF = 4
E_PAD = NC * NS * NCHUNK * CHUNK
ROWS_PER_TILE = N_PAD // NS

_F32 = jnp.float32


def _spmm_body(h_hbm, src_hbm, dst_hbm, out_hbm, src_v, dst_v, rows_v, zrow_v,
               acc, sem):
    c = lax.axis_index("c")
    s = lax.axis_index("s")
    pltpu.sync_copy(src_hbm.at[c, s], src_v)
    pltpu.sync_copy(dst_hbm.at[c, s], dst_v)
    zero16 = jnp.zeros((16,), _F32)

    def _z(i, carry):
        zrow_v[i, pl.ds(0, 16)] = zero16
        zrow_v[i, pl.ds(16, 16)] = zero16
        return carry

    lax.fori_loop(0, CHUNK, _z, 0)
    for k in range(ROWS_PER_TILE // CHUNK):
        pltpu.sync_copy(zrow_v, acc.at[pl.ds(s * ROWS_PER_TILE + k * CHUNK, CHUNK)])
    plsc.subcore_barrier()

    for b in range(NBUF):
        pltpu.async_copy(h_hbm.at[src_v.at[b]], rows_v.at[b], sem[b])

    def _grp(g, carry):
        for b in range(NBUF):
            j = g * NBUF + b
            pltpu.make_async_copy(h_hbm.at[src_v.at[0]], rows_v.at[b],
                                  sem[b]).wait()
            pltpu.sync_copy(rows_v.at[b], acc.at[dst_v.at[j]], add=True)

            @pl.when(j + NBUF < NCHUNK)
            def _():
                pltpu.async_copy(h_hbm.at[src_v.at[j + NBUF]], rows_v.at[b],
                                 sem[b])
        return carry

    lax.fori_loop(0, NCHUNK // NBUF, _grp, 0)
    plsc.subcore_barrier()
    pltpu.sync_copy(acc.at[pl.ds(s * ROWS_PER_TILE, ROWS_PER_TILE)],
                    out_hbm.at[c, pl.ds(s * ROWS_PER_TILE, ROWS_PER_TILE)])


def _deg_body(dst_hbm, out_hbm, dst_v, ones_v, zbuf_v, acc, sem):
    c = lax.axis_index("c")
    s = lax.axis_index("s")
    pltpu.sync_copy(dst_hbm.at[c, s], dst_v)
    one16 = jnp.full((16,), 1.0, _F32)
    for i in range(CHUNK // 16):
        ones_v[pl.ds(i * 16, 16)] = one16
    zero16 = jnp.zeros((16,), _F32)

    def _z(i, carry):
        zbuf_v[pl.ds(i * 16, 16)] = zero16
        return carry

    lax.fori_loop(0, ROWS_PER_TILE // 16, _z, 0)
    pltpu.sync_copy(zbuf_v, acc.at[pl.ds(s * ROWS_PER_TILE, ROWS_PER_TILE)])
    plsc.subcore_barrier()

    def _edge_chunk(j, carry):
        pltpu.sync_copy(ones_v, acc.at[dst_v.at[j]], add=True)
        return carry

    lax.fori_loop(0, NCHUNK, _edge_chunk, 0)
    plsc.subcore_barrier()
    pltpu.sync_copy(acc.at[pl.ds(s * ROWS_PER_TILE, ROWS_PER_TILE)],
                    out_hbm.at[c, pl.ds(s * ROWS_PER_TILE, ROWS_PER_TILE)])


def _make_spmm():
    mesh = plsc.VectorSubcoreMesh(core_axis_name="c", subcore_axis_name="s")
    return pl.kernel(
        _spmm_body,
        out_type=jax.ShapeDtypeStruct((NC, N_PAD, HID), _F32),
        mesh=mesh,
        scratch_types=[
            pltpu.VMEM((NCHUNK, CHUNK), jnp.int32),
            pltpu.VMEM((NCHUNK, CHUNK), jnp.int32),
            pltpu.VMEM((NBUF, CHUNK, HID), _F32),
            pltpu.VMEM((CHUNK, HID), _F32),
            pltpu.VMEM_SHARED((N_PAD, HID), _F32),
            [pltpu.SemaphoreType.DMA] * NBUF,
        ],
        compiler_params=pltpu.CompilerParams(use_tc_tiling_on_sc=False),
        name="gcn_spmm_sc",
    )


def _make_deg():
    mesh = plsc.VectorSubcoreMesh(core_axis_name="c", subcore_axis_name="s")
    return pl.kernel(
        _deg_body,
        out_type=jax.ShapeDtypeStruct((NC, N_PAD), _F32),
        mesh=mesh,
        scratch_types=[
            pltpu.VMEM((NCHUNK, CHUNK), jnp.int32),
            pltpu.VMEM((CHUNK,), _F32),
            pltpu.VMEM((ROWS_PER_TILE,), _F32),
            pltpu.VMEM_SHARED((N_PAD,), _F32),
            pltpu.SemaphoreType.DMA,
        ],
        compiler_params=pltpu.CompilerParams(use_tc_tiling_on_sc=False),
        name="gcn_deg_sc",
    )


_BLK = 1280


def _tc0_body(deg_ref, x_ref, w_ref, dinv_ref, h_ref):
    deg = deg_ref[:, 0:1] + deg_ref[:, 1:2] + 1.0
    dinv = lax.rsqrt(deg)
    dinv32 = jnp.broadcast_to(dinv, (dinv.shape[0], HID))
    dinv_ref[...] = dinv32
    h_ref[...] = dinv32 * jnp.dot(x_ref[...], w_ref[...],
                                  preferred_element_type=_F32)


def _tc_mid_body(agg_ref, h_ref, dinv_ref, w_ref, b_ref, out_ref):
    a = agg_ref[0] + agg_ref[1] + h_ref[...]
    xn = jnp.maximum(dinv_ref[...] * a + b_ref[...], 0.0)
    out_ref[...] = dinv_ref[...] * jnp.dot(xn, w_ref[...],
                                           preferred_element_type=_F32)


def _tc_last_body(agg_ref, h_ref, dinv_ref, w_ref, b_ref, out_ref):
    a = dinv_ref[...] * (agg_ref[0] + agg_ref[1] + h_ref[...])
    out_ref[...] = jnp.dot(a, w_ref[...], preferred_element_type=_F32) + b_ref[...]


def _tc0(degT, x_pad, W1):
    grid = (N_PAD // _BLK,)
    return pl.pallas_call(
        _tc0_body,
        grid=grid,
        in_specs=[
            pl.BlockSpec((_BLK, NC), lambda i: (i, 0)),
            pl.BlockSpec((_BLK, IN_DIM), lambda i: (i, 0)),
            pl.BlockSpec((IN_DIM, HID), lambda i: (0, 0)),
        ],
        out_specs=[
            pl.BlockSpec((_BLK, HID), lambda i: (i, 0)),
            pl.BlockSpec((_BLK, HID), lambda i: (i, 0)),
        ],
        out_shape=[
            jax.ShapeDtypeStruct((N_PAD, HID), _F32),
            jax.ShapeDtypeStruct((N_PAD, HID), _F32),
        ],
        name="gcn_tc0",
    )(degT, x_pad, W1)


def _tc_mid(agg, h, dinv32, W, b):
    grid = (N_PAD // _BLK,)
    return pl.pallas_call(
        _tc_mid_body,
        grid=grid,
        in_specs=[
            pl.BlockSpec((NC, _BLK, HID), lambda i: (0, i, 0)),
            pl.BlockSpec((_BLK, HID), lambda i: (i, 0)),
            pl.BlockSpec((_BLK, HID), lambda i: (i, 0)),
            pl.BlockSpec((HID, HID), lambda i: (0, 0)),
            pl.BlockSpec((1, HID), lambda i: (0, 0)),
        ],
        out_specs=pl.BlockSpec((_BLK, HID), lambda i: (i, 0)),
        out_shape=jax.ShapeDtypeStruct((N_PAD, HID), _F32),
        name="gcn_tc_mid",
    )(agg, h, dinv32, W, b)


def _tc_last(agg, h, dinv32, W4, b4):
    grid = (N_PAD // _BLK,)
    return pl.pallas_call(
        _tc_last_body,
        grid=grid,
        in_specs=[
            pl.BlockSpec((NC, _BLK, HID), lambda i: (0, i, 0)),
            pl.BlockSpec((_BLK, HID), lambda i: (i, 0)),
            pl.BlockSpec((_BLK, HID), lambda i: (i, 0)),
            pl.BlockSpec((HID, OUT_DIM), lambda i: (0, 0)),
            pl.BlockSpec((1, OUT_DIM), lambda i: (0, 0)),
        ],
        out_specs=pl.BlockSpec((_BLK, OUT_DIM), lambda i: (i, 0)),
        out_shape=jax.ShapeDtypeStruct((N_PAD, OUT_DIM), _F32),
        name="gcn_tc_last",
    )(agg, h, dinv32, W4, b4)


def kernel(x, edge_index, W1, b1, W2, b2, W3, b3, W4, b4):
    src = edge_index[0].astype(jnp.int32)
    dst = edge_index[1].astype(jnp.int32)
    pad = E_PAD - src.shape[0]
    src_p = jnp.concatenate([src, jnp.zeros((pad,), jnp.int32)])
    dst_p = jnp.concatenate([dst, jnp.full((pad,), N, jnp.int32)])
    src_p = src_p.reshape(NC, NS, NCHUNK, CHUNK)
    dst_p = dst_p.reshape(NC, NS, NCHUNK, CHUNK)
    x_pad = jnp.pad(x, ((0, N_PAD - N), (0, 0)))

    spmm = _make_spmm()
    degp = _make_deg()(dst_p)
    degT = degp.T

    dinv32, h1 = _tc0(degT, x_pad, W1)
    agg1 = spmm(h1, src_p, dst_p)
    h2 = _tc_mid(agg1, h1, dinv32, W2, b1.reshape(1, HID))
    agg2 = spmm(h2, src_p, dst_p)
    h3 = _tc_mid(agg2, h2, dinv32, W3, b2.reshape(1, HID))
    agg3 = spmm(h3, src_p, dst_p)
    x4 = _tc_mid(agg3, h3, dinv32, jnp.eye(HID, dtype=_F32),
                 b3.reshape(1, HID))
    agg4 = spmm(x4, src_p, dst_p)
    out = _tc_last(agg4, x4, dinv32, W4, b4.reshape(1, OUT_DIM))
    return out[:N]
